# Initial kernel scaffold; baseline (speedup 1.0000x reference)
import jax
import jax.numpy as jnp
from jax.experimental import pallas as pl


def kernel(x, edge_index, W1, b1, W2, b2):
    raise NotImplementedError("write your pallas kernel here")


# speedup vs baseline: 13.9025x; 13.9025x over previous
import functools

import jax
import jax.numpy as jnp
from jax import lax
from jax.experimental import pallas as pl
from jax.experimental.pallas import tpu as pltpu
from jax.experimental.pallas import tpu_sc as plsc

NC = 2
NS = 16
NW = NC * NS


def _deg_partials(dst, n_nodes):
    e = dst.shape[0]
    epw = e // NW
    ck = 80
    mesh = plsc.VectorSubcoreMesh(core_axis_name="c", subcore_axis_name="s")

    @functools.partial(
        pl.kernel,
        out_type=jax.ShapeDtypeStruct((NC, n_nodes), jnp.float32),
        mesh=mesh,
        scratch_types=[
            pltpu.VMEM((ck,), jnp.int32),
            pltpu.VMEM((ck,), jnp.float32),
            pltpu.VMEM_SHARED((n_nodes,), jnp.float32),
        ],
    )
    def k(dst_h, zeros_h, ones_h, out_h, idx_v, ones_v, deg_sh):
        cid = lax.axis_index("c")
        sid = lax.axis_index("s")
        base = (cid * NS + sid) * epw

        @pl.when(sid == 0)
        def _():
            pltpu.sync_copy(zeros_h, deg_sh)

        pltpu.sync_copy(ones_h, ones_v)
        plsc.subcore_barrier()

        def body(j, carry):
            pltpu.sync_copy(dst_h.at[pl.ds(base + j * ck, ck)], idx_v)
            pltpu.sync_copy(ones_v, deg_sh.at[idx_v], add=True)
            return carry

        lax.fori_loop(0, epw // ck, body, 0)
        plsc.subcore_barrier()

        @pl.when(sid == 0)
        def _():
            pltpu.sync_copy(deg_sh, out_h.at[cid])

    return k(dst, jnp.zeros((n_nodes,), jnp.float32), jnp.ones((ck,), jnp.float32))


def _propagate_partials(y, src, dst):
    n, d = y.shape
    e = src.shape[0]
    epw = e // NW
    ck = 80
    rows_per_tile = -(n // -NS) + 7 & ~7
    mesh = plsc.VectorSubcoreMesh(core_axis_name="c", subcore_axis_name="s")

    @functools.partial(
        pl.kernel,
        out_type=jax.ShapeDtypeStruct((NC, n, d), jnp.float32),
        mesh=mesh,
        scratch_types=[
            pltpu.VMEM((ck,), jnp.int32),
            pltpu.VMEM((ck,), jnp.int32),
            pltpu.VMEM((ck, d), jnp.float32),
            pltpu.VMEM_SHARED((n, d), jnp.float32),
            pltpu.SemaphoreType.DMA,
        ],
        compiler_params=pltpu.CompilerParams(use_tc_tiling_on_sc=False),
    )
    def k(y_h, src_h, dst_h, zeros_h, out_h, src_v, dst_v, rows_v, acc_sh, sem):
        cid = lax.axis_index("c")
        sid = lax.axis_index("s")
        base = (cid * NS + sid) * epw

        r0 = jnp.minimum(sid * rows_per_tile, n - rows_per_tile)
        pltpu.sync_copy(zeros_h.at[pl.ds(r0, rows_per_tile)],
                        acc_sh.at[pl.ds(r0, rows_per_tile)])
        plsc.subcore_barrier()

        def body(j, carry):
            off = base + j * ck
            pltpu.sync_copy(src_h.at[pl.ds(off, ck)], src_v)
            pltpu.sync_copy(dst_h.at[pl.ds(off, ck)], dst_v)
            pltpu.async_copy(y_h.at[src_v], rows_v, sem).wait()
            pltpu.sync_copy(rows_v, acc_sh.at[dst_v], add=True)
            return carry

        lax.fori_loop(0, epw // ck, body, 0)
        plsc.subcore_barrier()
        pltpu.sync_copy(acc_sh.at[pl.ds(r0, rows_per_tile)],
                        out_h.at[cid, pl.ds(r0, rows_per_tile)])

    return k(y, src, dst, jnp.zeros((n, d), jnp.float32))


_BR = 1000


def _tc_scale_in(x, w, deg_t):
    n, din = x.shape
    dout = w.shape[1]

    def body(x_ref, w_ref, dg_ref, y_ref, dis_ref):
        deg = dg_ref[:, 0:1] + dg_ref[:, 1:2] + 1.0
        dis = lax.rsqrt(deg)
        xw = jnp.dot(x_ref[...], w_ref[...], preferred_element_type=jnp.float32)
        y_ref[...] = xw * dis
        dis_ref[...] = dis

    return pl.pallas_call(
        body,
        grid=(n // _BR,),
        in_specs=[
            pl.BlockSpec((_BR, din), lambda i: (i, 0)),
            pl.BlockSpec((din, dout), lambda i: (0, 0)),
            pl.BlockSpec((_BR, 2), lambda i: (i, 0)),
        ],
        out_specs=[
            pl.BlockSpec((_BR, dout), lambda i: (i, 0)),
            pl.BlockSpec((_BR, 1), lambda i: (i, 0)),
        ],
        out_shape=[
            jax.ShapeDtypeStruct((n, dout), jnp.float32),
            jax.ShapeDtypeStruct((n, 1), jnp.float32),
        ],
    )(x, w, deg_t)


def _tc_mid(p, y1, dis, b1, w2):
    n, dh = y1.shape
    dout = w2.shape[1]

    def body(p_ref, y_ref, dis_ref, b_ref, w_ref, o_ref):
        s = p_ref[0] + p_ref[1] + y_ref[...]
        h = jnp.maximum(s * dis_ref[...] + b_ref[...], 0.0)
        o_ref[...] = jnp.dot(h, w_ref[...],
                             preferred_element_type=jnp.float32) * dis_ref[...]

    return pl.pallas_call(
        body,
        grid=(n // _BR,),
        in_specs=[
            pl.BlockSpec((NC, _BR, dh), lambda i: (0, i, 0)),
            pl.BlockSpec((_BR, dh), lambda i: (i, 0)),
            pl.BlockSpec((_BR, 1), lambda i: (i, 0)),
            pl.BlockSpec((1, dh), lambda i: (0, 0)),
            pl.BlockSpec((dh, dout), lambda i: (0, 0)),
        ],
        out_specs=pl.BlockSpec((_BR, dout), lambda i: (i, 0)),
        out_shape=jax.ShapeDtypeStruct((n, dout), jnp.float32),
    )(p, y1, dis, b1.reshape(1, dh), w2)


def _tc_out(p, y2, dis, b2):
    n, dout = y2.shape

    def body(p_ref, y_ref, dis_ref, b_ref, o_ref):
        o = (p_ref[0] + p_ref[1] + y_ref[...]) * dis_ref[...] + b_ref[...]
        m = jnp.max(o, axis=-1, keepdims=True)
        ex = jnp.exp(o - m)
        o_ref[...] = ex / jnp.sum(ex, axis=-1, keepdims=True)

    return pl.pallas_call(
        body,
        grid=(n // _BR,),
        in_specs=[
            pl.BlockSpec((NC, _BR, dout), lambda i: (0, i, 0)),
            pl.BlockSpec((_BR, dout), lambda i: (i, 0)),
            pl.BlockSpec((_BR, 1), lambda i: (i, 0)),
            pl.BlockSpec((1, dout), lambda i: (0, 0)),
        ],
        out_specs=pl.BlockSpec((_BR, dout), lambda i: (i, 0)),
        out_shape=jax.ShapeDtypeStruct((n, dout), jnp.float32),
    )(p, y2, dis, b2.reshape(1, dout))


def kernel(x, edge_index, W1, b1, W2, b2):
    n = x.shape[0]
    src = edge_index[0]
    dst = edge_index[1]

    degp = _deg_partials(dst, n)
    deg_t = jnp.transpose(degp)

    y1, dis = _tc_scale_in(x, W1, deg_t)
    p1 = _propagate_partials(y1, src, dst)
    y2 = _tc_mid(p1, y1, dis, b1, W2)
    p2 = _propagate_partials(y2, src, dst)
    return _tc_out(p2, y2, dis, b2)


# speedup vs baseline: 26.4305x; 1.9011x over previous
import functools

import jax
import jax.numpy as jnp
from jax import lax
from jax.experimental import pallas as pl
from jax.experimental.pallas import tpu as pltpu
from jax.experimental.pallas import tpu_sc as plsc

NC = 2
NS = 16
NW = NC * NS


def _deg_partials(dst, n_nodes):
    e = dst.shape[0]
    epw = e // NW
    ck = 80
    mesh = plsc.VectorSubcoreMesh(core_axis_name="c", subcore_axis_name="s")

    nchunk = epw // ck
    nq = 8

    @functools.partial(
        pl.kernel,
        out_type=jax.ShapeDtypeStruct((NC, n_nodes), jnp.float32),
        mesh=mesh,
        scratch_types=[
            pltpu.VMEM((nchunk, ck), jnp.int32),
            pltpu.VMEM((ck,), jnp.float32),
            pltpu.VMEM_SHARED((n_nodes,), jnp.float32),
            pltpu.SemaphoreType.DMA,
        ],
        compiler_params=pltpu.CompilerParams(use_tc_tiling_on_sc=False),
    )
    def k(dst_h, zeros_h, ones_h, out_h, idx_v, ones_v, deg_sh, sem):
        cid = lax.axis_index("c")
        sid = lax.axis_index("s")
        wid = cid * NS + sid

        @pl.when(sid == 0)
        def _():
            pltpu.sync_copy(zeros_h, deg_sh)

        pltpu.sync_copy(dst_h.at[pl.ds(wid * nchunk, nchunk)], idx_v)
        pltpu.sync_copy(ones_h, ones_v)
        plsc.subcore_barrier()

        def body(j, carry):
            pltpu.async_copy(ones_v, deg_sh.at[idx_v.at[j]], sem, add=True)

            @pl.when(j >= nq)
            def _():
                pltpu.make_async_copy(
                    ones_v, deg_sh.at[idx_v.at[0]], sem).wait()

            return carry

        lax.fori_loop(0, nchunk, body, 0)
        for _ in range(nq):
            pltpu.make_async_copy(ones_v, deg_sh.at[idx_v.at[0]], sem).wait()
        plsc.subcore_barrier()

        @pl.when(sid == 0)
        def _():
            pltpu.sync_copy(deg_sh, out_h.at[cid])

    return k(dst.reshape(NW * nchunk, ck), jnp.zeros((n_nodes,), jnp.float32),
             jnp.ones((ck,), jnp.float32))


def _propagate_partials(y, src, dst):
    n, d = y.shape
    e = src.shape[0]
    epw = e // NW
    ck = 80
    nchunk = epw // ck
    rows_per_tile = -(n // -NS) + 7 & ~7
    mesh = plsc.VectorSubcoreMesh(core_axis_name="c", subcore_axis_name="s")

    src2 = src.reshape(NW * nchunk, ck)
    dst2 = dst.reshape(NW * nchunk, ck)

    @functools.partial(
        pl.kernel,
        out_type=jax.ShapeDtypeStruct((NC, n, d), jnp.float32),
        mesh=mesh,
        scratch_types=[
            pltpu.VMEM((nchunk, ck), jnp.int32),
            pltpu.VMEM((nchunk, ck), jnp.int32),
            pltpu.VMEM((ck, d), jnp.float32),
            pltpu.VMEM((ck, d), jnp.float32),
            pltpu.VMEM_SHARED((n, d), jnp.float32),
            pltpu.SemaphoreType.DMA,
            pltpu.SemaphoreType.DMA,
            pltpu.SemaphoreType.DMA,
            pltpu.SemaphoreType.DMA,
        ],
        compiler_params=pltpu.CompilerParams(use_tc_tiling_on_sc=False),
    )
    def k(y_h, src_h, dst_h, zeros_h, out_h, srci_v, dsti_v, rows0, rows1,
          acc_sh, semg0, semg1, sems0, sems1):
        cid = lax.axis_index("c")
        sid = lax.axis_index("s")
        wid = cid * NS + sid

        pltpu.sync_copy(src_h.at[pl.ds(wid * nchunk, nchunk)], srci_v)
        pltpu.sync_copy(dst_h.at[pl.ds(wid * nchunk, nchunk)], dsti_v)
        r0 = jnp.minimum(sid * rows_per_tile, n - rows_per_tile)
        pltpu.sync_copy(zeros_h.at[pl.ds(r0, rows_per_tile)],
                        acc_sh.at[pl.ds(r0, rows_per_tile)])
        plsc.subcore_barrier()

        def step(j, cur, semg_cur, sems_cur, nxt, semg_nxt, sems_nxt):
            pltpu.make_async_copy(y_h.at[srci_v.at[j]], cur, semg_cur).wait()

            @pl.when(j + 1 < nchunk)
            def _():
                @pl.when(j >= 1)
                def _():
                    pltpu.make_async_copy(
                        nxt, acc_sh.at[dsti_v.at[j - 1]], sems_nxt).wait()
                pltpu.async_copy(y_h.at[srci_v.at[j + 1]], nxt, semg_nxt)

            pltpu.async_copy(cur, acc_sh.at[dsti_v.at[j]], sems_cur, add=True)

        pltpu.async_copy(y_h.at[srci_v.at[0]], rows0, semg0)

        def body(j, carry):
            @pl.when((j % 2) == 0)
            def _():
                step(j, rows0, semg0, sems0, rows1, semg1, sems1)

            @pl.when((j % 2) == 1)
            def _():
                step(j, rows1, semg1, sems1, rows0, semg0, sems0)

            return carry

        lax.fori_loop(0, nchunk, body, 0)
        pltpu.make_async_copy(rows0, acc_sh.at[dsti_v.at[0]], sems0).wait()
        pltpu.make_async_copy(rows1, acc_sh.at[dsti_v.at[0]], sems1).wait()

        plsc.subcore_barrier()
        pltpu.sync_copy(acc_sh.at[pl.ds(r0, rows_per_tile)],
                        out_h.at[cid, pl.ds(r0, rows_per_tile)])

    return k(y, src2, dst2, jnp.zeros((n, d), jnp.float32))


_BR = 1000


def _tc_scale_in(x, w, deg_t):
    n, din = x.shape
    dout = w.shape[1]

    def body(x_ref, w_ref, dg_ref, y_ref, dis_ref):
        deg = dg_ref[:, 0:1] + dg_ref[:, 1:2] + 1.0
        dis = lax.rsqrt(deg)
        xw = jnp.dot(x_ref[...], w_ref[...], preferred_element_type=jnp.float32)
        y_ref[...] = xw * dis
        dis_ref[...] = dis

    return pl.pallas_call(
        body,
        grid=(n // _BR,),
        in_specs=[
            pl.BlockSpec((_BR, din), lambda i: (i, 0)),
            pl.BlockSpec((din, dout), lambda i: (0, 0)),
            pl.BlockSpec((_BR, 2), lambda i: (i, 0)),
        ],
        out_specs=[
            pl.BlockSpec((_BR, dout), lambda i: (i, 0)),
            pl.BlockSpec((_BR, 1), lambda i: (i, 0)),
        ],
        out_shape=[
            jax.ShapeDtypeStruct((n, dout), jnp.float32),
            jax.ShapeDtypeStruct((n, 1), jnp.float32),
        ],
    )(x, w, deg_t)


def _tc_mid(p, y1, dis, b1, w2):
    n, dh = y1.shape
    dout = w2.shape[1]

    def body(p_ref, y_ref, dis_ref, b_ref, w_ref, o_ref):
        s = p_ref[0] + p_ref[1] + y_ref[...]
        h = jnp.maximum(s * dis_ref[...] + b_ref[...], 0.0)
        o_ref[...] = jnp.dot(h, w_ref[...],
                             preferred_element_type=jnp.float32) * dis_ref[...]

    return pl.pallas_call(
        body,
        grid=(n // _BR,),
        in_specs=[
            pl.BlockSpec((NC, _BR, dh), lambda i: (0, i, 0)),
            pl.BlockSpec((_BR, dh), lambda i: (i, 0)),
            pl.BlockSpec((_BR, 1), lambda i: (i, 0)),
            pl.BlockSpec((1, dh), lambda i: (0, 0)),
            pl.BlockSpec((dh, dout), lambda i: (0, 0)),
        ],
        out_specs=pl.BlockSpec((_BR, dout), lambda i: (i, 0)),
        out_shape=jax.ShapeDtypeStruct((n, dout), jnp.float32),
    )(p, y1, dis, b1.reshape(1, dh), w2)


def _tc_out(p, y2, dis, b2):
    n, dout = y2.shape

    def body(p_ref, y_ref, dis_ref, b_ref, o_ref):
        o = (p_ref[0] + p_ref[1] + y_ref[...]) * dis_ref[...] + b_ref[...]
        m = jnp.max(o, axis=-1, keepdims=True)
        ex = jnp.exp(o - m)
        o_ref[...] = ex / jnp.sum(ex, axis=-1, keepdims=True)

    return pl.pallas_call(
        body,
        grid=(n // _BR,),
        in_specs=[
            pl.BlockSpec((NC, _BR, dout), lambda i: (0, i, 0)),
            pl.BlockSpec((_BR, dout), lambda i: (i, 0)),
            pl.BlockSpec((_BR, 1), lambda i: (i, 0)),
            pl.BlockSpec((1, dout), lambda i: (0, 0)),
        ],
        out_specs=pl.BlockSpec((_BR, dout), lambda i: (i, 0)),
        out_shape=jax.ShapeDtypeStruct((n, dout), jnp.float32),
    )(p, y2, dis, b2.reshape(1, dout))


def kernel(x, edge_index, W1, b1, W2, b2):
    n = x.shape[0]
    src = edge_index[0]
    dst = edge_index[1]

    degp = _deg_partials(dst, n)
    deg_t = jnp.transpose(degp)

    y1, dis = _tc_scale_in(x, W1, deg_t)
    p1 = _propagate_partials(y1, src, dst)
    y2 = _tc_mid(p1, y1, dis, b1, W2)
    p2 = _propagate_partials(y2, src, dst)
    return _tc_out(p2, y2, dis, b2)


# speedup vs baseline: 36.7253x; 1.3895x over previous
import functools

import jax
import jax.numpy as jnp
from jax import lax
from jax.experimental import pallas as pl
from jax.experimental.pallas import tpu as pltpu
from jax.experimental.pallas import tpu_sc as plsc

NC = 2
NS = 16
NW = NC * NS


def _deg_partials(dst, n_nodes):
    e = dst.shape[0]
    epw = e // NW
    ck = 80
    mesh = plsc.VectorSubcoreMesh(core_axis_name="c", subcore_axis_name="s")

    nchunk = epw // ck
    nq = 8

    @functools.partial(
        pl.kernel,
        out_type=jax.ShapeDtypeStruct((NC, n_nodes), jnp.float32),
        mesh=mesh,
        scratch_types=[
            pltpu.VMEM((nchunk, ck), jnp.int32),
            pltpu.VMEM((ck,), jnp.float32),
            pltpu.VMEM_SHARED((n_nodes,), jnp.float32),
            pltpu.SemaphoreType.DMA,
        ],
        compiler_params=pltpu.CompilerParams(use_tc_tiling_on_sc=False),
    )
    def k(dst_h, zeros_h, ones_h, out_h, idx_v, ones_v, deg_sh, sem):
        cid = lax.axis_index("c")
        sid = lax.axis_index("s")
        wid = cid * NS + sid

        @pl.when(sid == 0)
        def _():
            pltpu.sync_copy(zeros_h, deg_sh)

        pltpu.sync_copy(dst_h.at[pl.ds(wid * nchunk, nchunk)], idx_v)
        pltpu.sync_copy(ones_h, ones_v)
        plsc.subcore_barrier()

        def body(j, carry):
            pltpu.async_copy(ones_v, deg_sh.at[idx_v.at[j]], sem, add=True)

            @pl.when(j >= nq)
            def _():
                pltpu.make_async_copy(
                    ones_v, deg_sh.at[idx_v.at[0]], sem).wait()

            return carry

        lax.fori_loop(0, nchunk, body, 0)
        for _ in range(nq):
            pltpu.make_async_copy(ones_v, deg_sh.at[idx_v.at[0]], sem).wait()
        plsc.subcore_barrier()

        @pl.when(sid == 0)
        def _():
            pltpu.sync_copy(deg_sh, out_h.at[cid])

    return k(dst.reshape(NW * nchunk, ck), jnp.zeros((n_nodes,), jnp.float32),
             jnp.ones((ck,), jnp.float32))


def _propagate_partials(y, src, dst):
    n, d = y.shape
    e = src.shape[0]
    epw = e // NW
    ck = 80
    nchunk = epw // ck
    nbuf = 3
    rows_per_tile = -(n // -NS) + 7 & ~7
    mesh = plsc.VectorSubcoreMesh(core_axis_name="c", subcore_axis_name="s")

    src2 = src.reshape(NW * nchunk, ck)
    dst2 = dst.reshape(NW * nchunk, ck)

    @functools.partial(
        pl.kernel,
        out_type=jax.ShapeDtypeStruct((NC, n, d), jnp.float32),
        mesh=mesh,
        scratch_types=[
            pltpu.VMEM((nchunk, ck), jnp.int32),
            pltpu.VMEM((nchunk, ck), jnp.int32),
            [pltpu.VMEM((ck, d), jnp.float32)] * nbuf,
            pltpu.VMEM_SHARED((n, d), jnp.float32),
            [pltpu.SemaphoreType.DMA] * nbuf,
            [pltpu.SemaphoreType.DMA] * nbuf,
        ],
        compiler_params=pltpu.CompilerParams(use_tc_tiling_on_sc=False),
    )
    def k(y_h, src_h, dst_h, zeros_h, out_h, srci_v, dsti_v, rows,
          acc_sh, semg, sems):
        cid = lax.axis_index("c")
        sid = lax.axis_index("s")
        wid = cid * NS + sid

        pltpu.sync_copy(src_h.at[pl.ds(wid * nchunk, nchunk)], srci_v)
        pltpu.sync_copy(dst_h.at[pl.ds(wid * nchunk, nchunk)], dsti_v)
        r0 = jnp.minimum(sid * rows_per_tile, n - rows_per_tile)
        pltpu.sync_copy(zeros_h.at[pl.ds(r0, rows_per_tile)],
                        acc_sh.at[pl.ds(r0, rows_per_tile)])
        plsc.subcore_barrier()

        def step(j, b):
            pltpu.make_async_copy(y_h.at[srci_v.at[j]], rows[b],
                                  semg[b]).wait()
            bn = (b + nbuf - 1) % nbuf

            @pl.when(j + nbuf - 1 < nchunk)
            def _():
                @pl.when(j >= 1)
                def _():
                    pltpu.make_async_copy(
                        rows[bn], acc_sh.at[dsti_v.at[0]], sems[bn]).wait()
                pltpu.async_copy(y_h.at[srci_v.at[j + nbuf - 1]], rows[bn],
                                 semg[bn])

            pltpu.async_copy(rows[b], acc_sh.at[dsti_v.at[j]], sems[b],
                             add=True)

        for b in range(nbuf - 1):
            pltpu.async_copy(y_h.at[srci_v.at[b]], rows[b], semg[b])

        def body(j, carry):
            for b in range(nbuf):
                @pl.when((j % nbuf) == b)
                def _(b=b):
                    step(j, b)
            return carry

        lax.fori_loop(0, nchunk, body, 0)
        for b in range(nbuf):
            pltpu.make_async_copy(rows[b], acc_sh.at[dsti_v.at[0]],
                                  sems[b]).wait()

        plsc.subcore_barrier()
        pltpu.sync_copy(acc_sh.at[pl.ds(r0, rows_per_tile)],
                        out_h.at[cid, pl.ds(r0, rows_per_tile)])

    return k(y, src2, dst2, jnp.zeros((n, d), jnp.float32))


_BR = 1000


def _tc_scale_in(x, w, deg_t):
    n, din = x.shape
    dout = w.shape[1]

    def body(x_ref, w_ref, dg_ref, y_ref, dis_ref):
        deg = dg_ref[:, 0:1] + dg_ref[:, 1:2] + 1.0
        dis = lax.rsqrt(deg)
        xw = jnp.dot(x_ref[...], w_ref[...], preferred_element_type=jnp.float32)
        y_ref[...] = xw * dis
        dis_ref[...] = dis

    return pl.pallas_call(
        body,
        grid=(n // _BR,),
        in_specs=[
            pl.BlockSpec((_BR, din), lambda i: (i, 0)),
            pl.BlockSpec((din, dout), lambda i: (0, 0)),
            pl.BlockSpec((_BR, 2), lambda i: (i, 0)),
        ],
        out_specs=[
            pl.BlockSpec((_BR, dout), lambda i: (i, 0)),
            pl.BlockSpec((_BR, 1), lambda i: (i, 0)),
        ],
        out_shape=[
            jax.ShapeDtypeStruct((n, dout), jnp.float32),
            jax.ShapeDtypeStruct((n, 1), jnp.float32),
        ],
    )(x, w, deg_t)


def _tc_mid(p, y1, dis, b1, w2):
    n, dh = y1.shape
    dout = w2.shape[1]

    def body(p_ref, y_ref, dis_ref, b_ref, w_ref, o_ref):
        s = p_ref[0] + p_ref[1] + y_ref[...]
        h = jnp.maximum(s * dis_ref[...] + b_ref[...], 0.0)
        o_ref[...] = jnp.dot(h, w_ref[...],
                             preferred_element_type=jnp.float32) * dis_ref[...]

    return pl.pallas_call(
        body,
        grid=(n // _BR,),
        in_specs=[
            pl.BlockSpec((NC, _BR, dh), lambda i: (0, i, 0)),
            pl.BlockSpec((_BR, dh), lambda i: (i, 0)),
            pl.BlockSpec((_BR, 1), lambda i: (i, 0)),
            pl.BlockSpec((1, dh), lambda i: (0, 0)),
            pl.BlockSpec((dh, dout), lambda i: (0, 0)),
        ],
        out_specs=pl.BlockSpec((_BR, dout), lambda i: (i, 0)),
        out_shape=jax.ShapeDtypeStruct((n, dout), jnp.float32),
    )(p, y1, dis, b1.reshape(1, dh), w2)


def _tc_out(p, y2, dis, b2):
    n, dout = y2.shape

    def body(p_ref, y_ref, dis_ref, b_ref, o_ref):
        o = (p_ref[0] + p_ref[1] + y_ref[...]) * dis_ref[...] + b_ref[...]
        m = jnp.max(o, axis=-1, keepdims=True)
        ex = jnp.exp(o - m)
        o_ref[...] = ex / jnp.sum(ex, axis=-1, keepdims=True)

    return pl.pallas_call(
        body,
        grid=(n // _BR,),
        in_specs=[
            pl.BlockSpec((NC, _BR, dout), lambda i: (0, i, 0)),
            pl.BlockSpec((_BR, dout), lambda i: (i, 0)),
            pl.BlockSpec((_BR, 1), lambda i: (i, 0)),
            pl.BlockSpec((1, dout), lambda i: (0, 0)),
        ],
        out_specs=pl.BlockSpec((_BR, dout), lambda i: (i, 0)),
        out_shape=jax.ShapeDtypeStruct((n, dout), jnp.float32),
    )(p, y2, dis, b2.reshape(1, dout))


def kernel(x, edge_index, W1, b1, W2, b2):
    n = x.shape[0]
    src = edge_index[0]
    dst = edge_index[1]

    degp = _deg_partials(dst, n)
    deg_t = jnp.transpose(degp)

    y1, dis = _tc_scale_in(x, W1, deg_t)
    p1 = _propagate_partials(y1, src, dst)
    y2 = _tc_mid(p1, y1, dis, b1, W2)
    p2 = _propagate_partials(y2, src, dst)
    return _tc_out(p2, y2, dis, b2)
